# Initial kernel scaffold; baseline (speedup 1.0000x reference)
import jax
import jax.numpy as jnp
from jax import lax
from jax.experimental import pallas as pl
from jax.experimental.pallas import tpu as pltpu
from jax.experimental.pallas import tpu_sc as plsc

N, E, FEAT, HID = 4096, 32768, 256, 128
NS = 16
NW = 2 * NS
SCAN = E // NS
AGG = E // NW
CHUNK = 128


def _f32(*shape):
    return jax.ShapeDtypeStruct(shape, jnp.float32)


def _ln(t, g, b, eps=1e-5):
    mu = jnp.mean(t, axis=-1, keepdims=True)
    var = jnp.mean((t - mu) ** 2, axis=-1, keepdims=True)
    return (t - mu) / jnp.sqrt(var + eps) * g + b


def _gelu(t):
    return 0.5 * t * (1.0 + lax.erf(t * 0.7071067811865476))


def _elu(t):
    return jnp.where(t > 0, t, jnp.exp(t) - 1.0)


def _tc1_body(x_ref, w1_ref, b1_ref, g0_ref, bb0_ref, gw_ref, att_ref,
              h_ref, hw_ref, sc_ref):
    t = jnp.dot(x_ref[...], w1_ref[...], preferred_element_type=jnp.float32)
    h = _gelu(_ln(t + b1_ref[...], g0_ref[...], bb0_ref[...]))
    h_ref[...] = h
    hw = jnp.dot(h, gw_ref[...], preferred_element_type=jnp.float32)
    hw_ref[...] = hw
    sc_ref[...] = jnp.dot(hw, att_ref[...], preferred_element_type=jnp.float32)


_tc1 = pl.pallas_call(
    _tc1_body,
    out_shape=(_f32(N, HID), _f32(N, HID), _f32(N, 2)),
)


def _tc2_body(h_ref, parts_ref, g_ref, b_ref, gw_ref, att_ref,
              h1_ref, hw_ref, sc_ref):
    agg = parts_ref[0] + parts_ref[1]
    h1 = _ln(h_ref[...] + _elu(agg), g_ref[...], b_ref[...])
    h1_ref[...] = h1
    hw = jnp.dot(h1, gw_ref[...], preferred_element_type=jnp.float32)
    hw_ref[...] = hw
    sc_ref[...] = jnp.dot(hw, att_ref[...], preferred_element_type=jnp.float32)


_tc2 = pl.pallas_call(
    _tc2_body,
    out_shape=(_f32(N, HID), _f32(N, HID), _f32(N, 2)),
)


def _tc3_body(h1_ref, parts_ref, g_ref, b_ref, w1_ref, b1_ref, w2_ref,
              b2_ref, out_ref):
    agg = parts_ref[0] + parts_ref[1]
    h2 = _ln(h1_ref[...] + _elu(agg), g_ref[...], b_ref[...])
    gm = jnp.mean(h2, axis=0, keepdims=True)
    s1 = _gelu(jnp.dot(gm, w1_ref[...], preferred_element_type=jnp.float32)
               + b1_ref[...])
    out_ref[...] = (jnp.dot(s1, w2_ref[...], preferred_element_type=jnp.float32)
                    + b2_ref[...])


_tc3 = pl.pallas_call(_tc3_body, out_shape=_f32(1, 1))


def _sc_gat_body(ei_hbm, sc_hbm, hw_hbm, out_hbm,
                 sc_loc, den_loc, src_scan, dst_scan, ex_scan,
                 sidx, didx, exch, alpha, rows,
                 den_sh, ex_sh, agg_sh, sem):
    c = lax.axis_index("c")
    s = lax.axis_index("s")
    zeros16 = jnp.zeros((16,), jnp.float32)
    izeros16 = jnp.zeros((16,), jnp.int32)
    iones16 = jnp.ones((16,), jnp.int32)

    pltpu.sync_copy(sc_hbm, sc_loc)
    base_scan = s * SCAN
    pltpu.sync_copy(ei_hbm.at[0, pl.ds(base_scan, SCAN)], src_scan)
    pltpu.sync_copy(ei_hbm.at[1, pl.ds(base_scan, SCAN)], dst_scan)

    def zrow(r, _):
        for cc in range(HID // 16):
            rows[r, pl.ds(cc * 16, 16)] = zeros16
        return 0
    lax.fori_loop(0, CHUNK, zrow, 0)

    def zden(i, _):
        den_loc[pl.ds(i * 16, 16)] = zeros16
        return 0
    lax.fori_loop(0, N // 16, zden, 0)

    rows_per_sub = N // NS
    for j in range(rows_per_sub // CHUNK):
        pltpu.sync_copy(rows, agg_sh.at[pl.ds(s * rows_per_sub + j * CHUNK,
                                              CHUNK)])

    @pl.when(s == 0)
    def _():
        pltpu.sync_copy(den_loc, den_sh)

    plsc.subcore_barrier()

    def scal(i, _):
        o = i * 16
        sv = src_scan[pl.ds(o, 16)]
        dv = dst_scan[pl.ds(o, 16)]
        a = plsc.load_gather(sc_loc, [sv, izeros16])
        b = plsc.load_gather(sc_loc, [dv, iones16])
        e = a + b
        ex_scan[pl.ds(o, 16)] = jnp.exp(jnp.maximum(e, 0.2 * e))
        return 0
    lax.fori_loop(0, SCAN // 16, scal, 0)

    pltpu.sync_copy(ex_scan, ex_sh.at[pl.ds(base_scan, SCAN)])

    def dscat(k, _):
        pltpu.sync_copy(dst_scan.at[pl.ds(k * CHUNK, CHUNK)], didx)
        pltpu.sync_copy(ex_scan.at[pl.ds(k * CHUNK, CHUNK)], exch)
        pltpu.sync_copy(exch, den_sh.at[didx], add=True)
        return 0
    lax.fori_loop(0, SCAN // CHUNK, dscat, 0)

    plsc.subcore_barrier()

    pltpu.sync_copy(den_sh, den_loc)

    base_agg = (c * NS + s) * AGG

    def chunkfn(k, _):
        off = base_agg + k * CHUNK
        pltpu.sync_copy(ei_hbm.at[0, pl.ds(off, CHUNK)], sidx)
        pltpu.sync_copy(ei_hbm.at[1, pl.ds(off, CHUNK)], didx)
        pltpu.sync_copy(ex_sh.at[pl.ds(off, CHUNK)], exch)
        pltpu.async_copy(hw_hbm.at[sidx], rows, sem).wait()

        def av(i, _):
            o = i * 16
            dv = didx[pl.ds(o, 16)]
            den_g = plsc.load_gather(den_loc, [dv])
            alpha[pl.ds(o, 16)] = exch[pl.ds(o, 16)] / den_g
            return 0
        lax.fori_loop(0, CHUNK // 16, av, 0)

        def rv(r, _):
            asp = plsc.load_gather(alpha, [jnp.full((16,), r, jnp.int32)])
            for cc in range(HID // 16):
                rows[r, pl.ds(cc * 16, 16)] = rows[r, pl.ds(cc * 16, 16)] * asp
            return 0
        lax.fori_loop(0, CHUNK, rv, 0)

        pltpu.sync_copy(rows, agg_sh.at[didx], add=True)
        return 0
    lax.fori_loop(0, AGG // CHUNK, chunkfn, 0)

    plsc.subcore_barrier()

    for j in range(rows_per_sub // CHUNK):
        o = s * rows_per_sub + j * CHUNK
        pltpu.sync_copy(agg_sh.at[pl.ds(o, CHUNK)],
                        out_hbm.at[c, pl.ds(o, CHUNK)])


_sc_gat = pl.kernel(
    _sc_gat_body,
    out_type=_f32(2, N, HID),
    mesh=plsc.VectorSubcoreMesh(core_axis_name="c", subcore_axis_name="s"),
    scratch_types=[
        pltpu.VMEM((N, 2), jnp.float32),
        pltpu.VMEM((N,), jnp.float32),
        pltpu.VMEM((SCAN,), jnp.int32),
        pltpu.VMEM((SCAN,), jnp.int32),
        pltpu.VMEM((SCAN,), jnp.float32),
        pltpu.VMEM((CHUNK,), jnp.int32),
        pltpu.VMEM((CHUNK,), jnp.int32),
        pltpu.VMEM((CHUNK,), jnp.float32),
        pltpu.VMEM((CHUNK,), jnp.float32),
        pltpu.VMEM((CHUNK, HID), jnp.float32),
        pltpu.VMEM_SHARED((N,), jnp.float32),
        pltpu.VMEM_SHARED((E,), jnp.float32),
        pltpu.VMEM_SHARED((N, HID), jnp.float32),
        pltpu.SemaphoreType.DMA,
    ],
)


@jax.jit
def kernel(x, edge_index, edge_attr, W1, b1, ln0_g, ln0_b, edge_table,
           g1_W, g1_att, ln1_g, ln1_b, g2_W, g2_att, ln2_g, ln2_b,
           sh_W1, sh_b1, sh_W2, sh_b2):
    att1 = jnp.concatenate([g1_att[:HID], g1_att[HID:]], axis=1)
    att2 = jnp.concatenate([g2_att[:HID], g2_att[HID:]], axis=1)
    row = lambda v: v.reshape(1, -1)

    h, hw1, sc1 = _tc1(x, W1, row(b1), row(ln0_g), row(ln0_b), g1_W, att1)
    parts1 = _sc_gat(edge_index, sc1, hw1)
    h1, hw2, sc2 = _tc2(h, parts1, row(ln1_g), row(ln1_b), g2_W, att2)
    parts2 = _sc_gat(edge_index, sc2, hw2)
    out = _tc3(h1, parts2, row(ln2_g), row(ln2_b), sh_W1, row(sh_b1),
               sh_W2, row(sh_b2))
    return out.reshape(-1)


# speedup vs baseline: 12.5104x; 12.5104x over previous
import jax
import jax.numpy as jnp
from jax import lax
from jax.experimental import pallas as pl
from jax.experimental.pallas import tpu as pltpu
from jax.experimental.pallas import tpu_sc as plsc

N, E, FEAT, HID = 4096, 32768, 256, 128
NS = 16
NW = 2 * NS
SCAN = E // NS
AGG = E // NW
CHUNK = 128


def _f32(*shape):
    return jax.ShapeDtypeStruct(shape, jnp.float32)


def _ln(t, g, b, eps=1e-5):
    mu = jnp.mean(t, axis=-1, keepdims=True)
    var = jnp.mean((t - mu) ** 2, axis=-1, keepdims=True)
    return (t - mu) / jnp.sqrt(var + eps) * g + b


def _gelu(t):
    return 0.5 * t * (1.0 + lax.erf(t * 0.7071067811865476))


def _elu(t):
    return jnp.where(t > 0, t, jnp.exp(t) - 1.0)


def _tc1_body(x_ref, w1_ref, b1_ref, g0_ref, bb0_ref, gw_ref, att_ref,
              h_ref, hw_ref, sc_ref):
    t = jnp.dot(x_ref[...], w1_ref[...], preferred_element_type=jnp.float32)
    h = _gelu(_ln(t + b1_ref[...], g0_ref[...], bb0_ref[...]))
    h_ref[...] = h
    hw = jnp.dot(h, gw_ref[...], preferred_element_type=jnp.float32)
    hw_ref[...] = hw
    sc_ref[...] = jnp.dot(hw, att_ref[...], preferred_element_type=jnp.float32)


_tc1 = pl.pallas_call(
    _tc1_body,
    out_shape=(_f32(N, HID), _f32(N, HID), _f32(N, 2)),
)


def _tc2_body(h_ref, parts_ref, g_ref, b_ref, gw_ref, att_ref,
              h1_ref, hw_ref, sc_ref):
    agg = parts_ref[0] + parts_ref[1]
    h1 = _ln(h_ref[...] + _elu(agg), g_ref[...], b_ref[...])
    h1_ref[...] = h1
    hw = jnp.dot(h1, gw_ref[...], preferred_element_type=jnp.float32)
    hw_ref[...] = hw
    sc_ref[...] = jnp.dot(hw, att_ref[...], preferred_element_type=jnp.float32)


_tc2 = pl.pallas_call(
    _tc2_body,
    out_shape=(_f32(N, HID), _f32(N, HID), _f32(N, 2)),
)


def _tc3_body(h1_ref, parts_ref, g_ref, b_ref, w1_ref, b1_ref, w2_ref,
              b2_ref, out_ref):
    agg = parts_ref[0] + parts_ref[1]
    h2 = _ln(h1_ref[...] + _elu(agg), g_ref[...], b_ref[...])
    gm = jnp.mean(h2, axis=0, keepdims=True)
    s1 = _gelu(jnp.dot(gm, w1_ref[...], preferred_element_type=jnp.float32)
               + b1_ref[...])
    out_ref[...] = (jnp.dot(s1, w2_ref[...], preferred_element_type=jnp.float32)
                    + b2_ref[...])


_tc3 = pl.pallas_call(_tc3_body, out_shape=_f32(1, 1))


def _sc_gat_body(ei_hbm, sc_hbm, hw_hbm, out_hbm,
                 sc_loc, den_loc, src_scan, dst_scan, ex_scan,
                 sidx, didx, exch, alpha, rows,
                 den_sh, ex_sh, agg_sh, sem):
    c = lax.axis_index("c")
    s = lax.axis_index("s")
    zeros16 = jnp.zeros((16,), jnp.float32)
    iones16 = jnp.ones((16,), jnp.int32)

    pltpu.sync_copy(sc_hbm, sc_loc)
    base_scan = s * SCAN
    pltpu.sync_copy(ei_hbm.at[0, pl.ds(base_scan, SCAN)], src_scan)
    pltpu.sync_copy(ei_hbm.at[1, pl.ds(base_scan, SCAN)], dst_scan)

    def zrow(r, _):
        for cc in range(HID // 16):
            rows[r, pl.ds(cc * 16, 16)] = zeros16
        return 0
    lax.fori_loop(0, CHUNK, zrow, 0)

    def zden(i, _):
        den_loc[pl.ds(i * 16, 16)] = zeros16
        return 0
    lax.fori_loop(0, N // 16, zden, 0)

    rows_per_sub = N // NS
    for j in range(rows_per_sub // CHUNK):
        pltpu.sync_copy(rows, agg_sh.at[pl.ds(s * rows_per_sub + j * CHUNK,
                                              CHUNK)])

    @pl.when(s == 0)
    def _():
        pltpu.sync_copy(den_loc, den_sh)

    plsc.subcore_barrier()

    def scal(i, _):
        o = i * 16
        sv = src_scan[pl.ds(o, 16)]
        dv = dst_scan[pl.ds(o, 16)]
        a = plsc.load_gather(sc_loc, [sv * 2])
        b = plsc.load_gather(sc_loc, [dv * 2 + iones16])
        e = a + b
        ex_scan[pl.ds(o, 16)] = jnp.exp(jnp.maximum(e, 0.2 * e))
        return 0
    lax.fori_loop(0, SCAN // 16, scal, 0)

    pltpu.sync_copy(ex_scan, ex_sh.at[pl.ds(base_scan, SCAN)])

    def dscat(k, _):
        for j in range(CHUNK // 16):
            didx[pl.ds(j * 16, 16)] = dst_scan[pl.ds(k * CHUNK + j * 16, 16)]
            exch[pl.ds(j * 16, 16)] = ex_scan[pl.ds(k * CHUNK + j * 16, 16)]
        pltpu.sync_copy(exch, den_sh.at[didx], add=True)
        return 0
    lax.fori_loop(0, SCAN // CHUNK, dscat, 0)

    plsc.subcore_barrier()

    pltpu.sync_copy(den_sh, den_loc)

    base_agg = (c * NS + s) * AGG

    def chunkfn(k, _):
        off = base_agg + k * CHUNK
        pltpu.sync_copy(ei_hbm.at[0, pl.ds(off, CHUNK)], sidx)
        pltpu.sync_copy(ei_hbm.at[1, pl.ds(off, CHUNK)], didx)
        pltpu.sync_copy(ex_sh.at[pl.ds(off, CHUNK)], exch)
        pltpu.async_copy(hw_hbm.at[sidx], rows, sem).wait()

        def av(i, _):
            o = i * 16
            dv = didx[pl.ds(o, 16)]
            den_g = plsc.load_gather(den_loc, [dv])
            alpha[pl.ds(o, 16)] = exch[pl.ds(o, 16)] / den_g
            return 0
        lax.fori_loop(0, CHUNK // 16, av, 0)

        def rv(r, _):
            asp = plsc.load_gather(alpha, [jnp.full((16,), r, jnp.int32)])
            for cc in range(HID // 16):
                rows[r, pl.ds(cc * 16, 16)] = rows[r, pl.ds(cc * 16, 16)] * asp
            return 0
        lax.fori_loop(0, CHUNK, rv, 0)

        pltpu.sync_copy(rows, agg_sh.at[didx], add=True)
        return 0
    lax.fori_loop(0, AGG // CHUNK, chunkfn, 0)

    plsc.subcore_barrier()

    for j in range(rows_per_sub // CHUNK):
        o = s * rows_per_sub + j * CHUNK
        pltpu.sync_copy(agg_sh.at[pl.ds(o, CHUNK)],
                        out_hbm.at[c, pl.ds(o, CHUNK)])


_sc_gat = pl.kernel(
    _sc_gat_body,
    out_type=_f32(2, N, HID),
    mesh=plsc.VectorSubcoreMesh(core_axis_name="c", subcore_axis_name="s"),
    compiler_params=pltpu.CompilerParams(needs_layout_passes=False),
    scratch_types=[
        pltpu.VMEM((2 * N,), jnp.float32),
        pltpu.VMEM((N,), jnp.float32),
        pltpu.VMEM((SCAN,), jnp.int32),
        pltpu.VMEM((SCAN,), jnp.int32),
        pltpu.VMEM((SCAN,), jnp.float32),
        pltpu.VMEM((CHUNK,), jnp.int32),
        pltpu.VMEM((CHUNK,), jnp.int32),
        pltpu.VMEM((CHUNK,), jnp.float32),
        pltpu.VMEM((CHUNK,), jnp.float32),
        pltpu.VMEM((CHUNK, HID), jnp.float32),
        pltpu.VMEM_SHARED((N,), jnp.float32),
        pltpu.VMEM_SHARED((E,), jnp.float32),
        pltpu.VMEM_SHARED((N, HID), jnp.float32),
        pltpu.SemaphoreType.DMA,
    ],
)


@jax.jit
def kernel(x, edge_index, edge_attr, W1, b1, ln0_g, ln0_b, edge_table,
           g1_W, g1_att, ln1_g, ln1_b, g2_W, g2_att, ln2_g, ln2_b,
           sh_W1, sh_b1, sh_W2, sh_b2):
    att1 = jnp.concatenate([g1_att[:HID], g1_att[HID:]], axis=1)
    att2 = jnp.concatenate([g2_att[:HID], g2_att[HID:]], axis=1)
    row = lambda v: v.reshape(1, -1)

    h, hw1, sc1 = _tc1(x, W1, row(b1), row(ln0_g), row(ln0_b), g1_W, att1)
    parts1 = _sc_gat(edge_index, sc1.reshape(-1), hw1)
    h1, hw2, sc2 = _tc2(h, parts1, row(ln1_g), row(ln1_b), g2_W, att2)
    parts2 = _sc_gat(edge_index, sc2.reshape(-1), hw2)
    out = _tc3(h1, parts2, row(ln2_g), row(ln2_b), sh_W1, row(sh_b1),
               sh_W2, row(sh_b2))
    return out.reshape(-1)
